# Initial kernel scaffold; baseline (speedup 1.0000x reference)
import jax
import jax.numpy as jnp
from jax.experimental import pallas as pl


def kernel(anchor_emb, positive_emb, negative_embs, tree_distances, anchor_codes, positive_codes, negative_codes, batch_size, k_negatives):
    raise NotImplementedError("write your pallas kernel here")


# speedup vs baseline: 352.1598x; 352.1598x over previous
import functools

import jax
import jax.numpy as jnp
from jax import lax
from jax.experimental import pallas as pl
from jax.experimental.pallas import tpu as pltpu
from jax.experimental.pallas import tpu_sc as plsc

WEIGHT = 0.15
SIGMA = 1.0
NDCG_K = 10

B = 512
K = 31
N = K + 1
DSP = 128
V = 2048


_LANES = 16
_TOTAL = B * N


def _sc_gather(td_flat, fi):
    info = plsc.get_sparse_core_info()
    nw = info.num_cores * info.num_subcores
    per_w = _TOTAL // nw
    mesh = plsc.VectorSubcoreMesh(core_axis_name="c", subcore_axis_name="s")

    @functools.partial(
        pl.kernel,
        mesh=mesh,
        out_type=jax.ShapeDtypeStruct((_TOTAL,), jnp.float32),
        scratch_types=[
            pltpu.VMEM((per_w,), jnp.int32),
            pltpu.VMEM((per_w,), jnp.float32),
            pltpu.SemaphoreType.DMA,
        ],
    )
    def k(td_hbm, fi_hbm, out_hbm, fi_v, out_v, sem):
        wid = lax.axis_index("s") * info.num_cores + lax.axis_index("c")
        base = wid * per_w
        pltpu.sync_copy(fi_hbm.at[pl.ds(base, per_w)], fi_v)
        pltpu.async_copy(td_hbm.at[fi_v], out_v, sem).wait()
        pltpu.sync_copy(out_v, out_hbm.at[pl.ds(base, per_w)])

    return k(td_flat, fi)


_BB = 64


def _tc_body(a_ref, p_ref, n_ref, td_ref, out_ref):
    i = pl.program_id(0)
    a = a_ref[...]
    p = p_ref[...]
    nsp = n_ref[...]
    td = td_ref[...]

    at = jnp.sqrt(1.0 + jnp.sum(a * a, axis=-1, keepdims=True))
    pt = jnp.sqrt(1.0 + jnp.sum(p * p, axis=-1, keepdims=True))
    ip = jnp.sum(a * p, axis=-1, keepdims=True) - at * pt
    nt = jnp.sqrt(1.0 + jnp.sum(nsp * nsp, axis=-1))
    inn = jnp.sum(a[:, None, :] * nsp, axis=-1) - at * nt
    inner = jnp.concatenate([ip, inn], axis=1)
    z = jnp.maximum(-inner, 1.0 + 1e-7)
    dist = jnp.log(z + jnp.sqrt((z - 1.0) * (z + 1.0)))

    maxtd = jnp.max(td, axis=1, keepdims=True)
    rel = (maxtd - td + 1e-6) / (maxtd + 1e-6)

    ii = lax.broadcasted_iota(jnp.int32, (N, N), 0)
    jj = lax.broadcasted_iota(jnp.int32, (N, N), 1)
    tie = (jj < ii)[None]

    di = dist[:, :, None]
    dj = dist[:, None, :]
    ranks = jnp.sum(
        jnp.where((dj < di) | ((dj == di) & tie), 1, 0), axis=2
    ).astype(jnp.float32)
    g = jnp.where(ranks < NDCG_K, 1.0 / jnp.log2(ranks + 2.0), 0.0)

    ri2 = rel[:, :, None]
    rj2 = rel[:, None, :]
    rranks = jnp.sum(
        jnp.where((rj2 > ri2) | ((rj2 == ri2) & tie), 1, 0), axis=2
    ).astype(jnp.float32)
    rdisc = jnp.where(rranks < NDCG_K, 1.0 / jnp.log2(rranks + 2.0), 0.0)
    ideal = jnp.sum(rel * rdisc, axis=1, keepdims=True)

    gi = g[:, :, None]
    gj = g[:, None, :]
    delta = jnp.abs((ri2 - rj2) * (gj - gi)) / jnp.maximum(ideal, 1e-30)[..., None]
    delta = jnp.where(ideal[..., None] > 0.0, delta, 0.0)
    prob = 1.0 / (1.0 + jnp.exp(SIGMA * (dj - di)))
    lam = jnp.where(
        ri2 > rj2, delta * (1.0 - prob),
        jnp.where(rj2 > ri2, -delta * prob, 0.0),
    )
    upper = (ii < jj)[None]
    part = jnp.sum(jnp.where(upper, lam * (di - dj), 0.0)) * (WEIGHT / B)

    acc = jnp.where(i == 0, 0.0, out_ref[...])
    out_ref[...] = acc + jnp.full((1, 1), part, jnp.float32)


def _tc_compute(a_sp, p_sp, n_sp, all_td):
    grid = B // _BB
    return pl.pallas_call(
        _tc_body,
        grid=(grid,),
        in_specs=[
            pl.BlockSpec((_BB, DSP), lambda i: (i, 0)),
            pl.BlockSpec((_BB, DSP), lambda i: (i, 0)),
            pl.BlockSpec((_BB, K, DSP), lambda i: (i, 0, 0)),
            pl.BlockSpec((_BB, N), lambda i: (i, 0)),
        ],
        out_specs=pl.BlockSpec((1, 1), lambda i: (0, 0)),
        out_shape=jax.ShapeDtypeStruct((1, 1), jnp.float32),
    )(a_sp, p_sp, n_sp, all_td)


def kernel(anchor_emb, positive_emb, negative_embs, tree_distances,
           anchor_codes, positive_codes, negative_codes,
           batch_size, k_negatives):
    a_sp = anchor_emb[:, 1:]
    p_sp = positive_emb[:, 1:]
    n_sp = negative_embs[:, 1:].reshape(B, K, DSP)

    all_codes = jnp.concatenate(
        [positive_codes[:, None], negative_codes], axis=1)
    fi = (anchor_codes[:, None].astype(jnp.int32) * V
          + all_codes.astype(jnp.int32)).reshape(-1)
    td_flat = tree_distances.reshape(V * V)

    all_td = _sc_gather(td_flat, fi).reshape(B, N)
    out = _tc_compute(a_sp, p_sp, n_sp, all_td)
    return out[0, 0] + 0.0 * k_negatives


# speedup vs baseline: 569.9024x; 1.6183x over previous
import functools

import jax
import jax.numpy as jnp
from jax import lax
from jax.experimental import pallas as pl
from jax.experimental.pallas import tpu as pltpu
from jax.experimental.pallas import tpu_sc as plsc

WEIGHT = 0.15
SIGMA = 1.0
NDCG_K = 10

B = 512
K = 31
N = K + 1
NN = N * N
DE = 129
V = 2048


_TOTAL = B * N


def _sc_gather(td_flat, fi):
    info = plsc.get_sparse_core_info()
    nw = info.num_cores * info.num_subcores
    per_w = _TOTAL // nw
    mesh = plsc.VectorSubcoreMesh(core_axis_name="c", subcore_axis_name="s")

    @functools.partial(
        pl.kernel,
        mesh=mesh,
        out_type=jax.ShapeDtypeStruct((_TOTAL,), jnp.float32),
        scratch_types=[
            pltpu.VMEM((per_w,), jnp.int32),
            pltpu.VMEM((per_w,), jnp.float32),
            pltpu.SemaphoreType.DMA,
        ],
    )
    def k(td_hbm, fi_hbm, out_hbm, fi_v, out_v, sem):
        wid = lax.axis_index("s") * info.num_cores + lax.axis_index("c")
        base = wid * per_w
        pltpu.sync_copy(fi_hbm.at[pl.ds(base, per_w)], fi_v)
        pltpu.async_copy(td_hbm.at[fi_v], out_v, sem).wait()
        pltpu.sync_copy(out_v, out_hbm.at[pl.ds(base, per_w)])

    return k(td_flat, fi)


_BB = 64
_HI = jax.lax.Precision.HIGHEST


def _tc_body(a_ref, p_ref, n_ref, td_ref, out_ref):
    step = pl.program_id(0)
    a = a_ref[...]
    p = p_ref[...]
    nf = n_ref[...]
    td = td_ref[...]

    a0 = a[:, 0:1]
    p0 = p[:, 0:1]
    n0 = nf[:, :, 0]
    asq = jnp.sum(a * a, axis=-1, keepdims=True) - a0 * a0
    psq = jnp.sum(p * p, axis=-1, keepdims=True) - p0 * p0
    nsq = jnp.sum(nf * nf, axis=-1) - n0 * n0
    ta = jnp.sqrt(1.0 + asq)
    tp = jnp.sqrt(1.0 + psq)
    tn = jnp.sqrt(1.0 + nsq)
    ip = jnp.sum(a * p, axis=-1, keepdims=True) - a0 * p0 - ta * tp
    inn = jnp.sum(a[:, None, :] * nf, axis=-1) - a0 * n0 - ta * tn
    inner = jnp.concatenate([ip, inn], axis=1)
    z = jnp.maximum(-inner, 1.0 + 1e-7)
    dist = jnp.log(z + jnp.sqrt((z - 1.0) * (z + 1.0)))

    maxtd = jnp.max(td, axis=1, keepdims=True)
    rel = (maxtd - td + 1e-6) / (maxtd + 1e-6)

    row = lax.broadcasted_iota(jnp.int32, (N, NN), 0)
    colp = lax.broadcasted_iota(jnp.int32, (N, NN), 1)
    ei = (lax.shift_right_logical(colp, 5) == row).astype(jnp.float32)
    ej = ((colp & (N - 1)) == row).astype(jnp.float32)

    def bcast(x, e):
        return jax.lax.dot(x, e, precision=_HI)

    def pair_reduce(x, e):
        return lax.dot_general(x, e, (((1,), (1,)), ((), ())), precision=_HI)

    iip = lax.shift_right_logical(
        lax.broadcasted_iota(jnp.int32, (1, NN), 1), 5)
    jjp = lax.broadcasted_iota(jnp.int32, (1, NN), 1) & (N - 1)
    tie = (jjp < iip).astype(jnp.float32)
    upper = iip < jjp

    di = bcast(dist, ei)
    dj = bcast(dist, ej)
    lt = jnp.where(dj < di, 1.0, 0.0) + jnp.where(dj == di, tie, 0.0)
    ranks = pair_reduce(lt, ei)
    g = jnp.where(ranks < NDCG_K, 1.0 / jnp.log2(ranks + 2.0), 0.0)

    ri = bcast(rel, ei)
    rj = bcast(rel, ej)
    rlt = jnp.where(rj > ri, 1.0, 0.0) + jnp.where(rj == ri, tie, 0.0)
    rranks = pair_reduce(rlt, ei)
    rdisc = jnp.where(rranks < NDCG_K, 1.0 / jnp.log2(rranks + 2.0), 0.0)
    ideal = jnp.sum(rel * rdisc, axis=1, keepdims=True)

    gi = bcast(g, ei)
    gj = bcast(g, ej)
    delta = jnp.abs((ri - rj) * (gj - gi)) / jnp.maximum(ideal, 1e-30)
    delta = jnp.where(ideal > 0.0, delta, 0.0)
    prob = 1.0 / (1.0 + jnp.exp(SIGMA * (dj - di)))
    lam = jnp.where(
        ri > rj, delta * (1.0 - prob),
        jnp.where(rj > ri, -delta * prob, 0.0),
    )
    part = jnp.sum(jnp.where(upper, lam * (di - dj), 0.0)) * (WEIGHT / B)

    acc = jnp.where(step == 0, 0.0, out_ref[...])
    out_ref[...] = acc + jnp.full((1, 1), part, jnp.float32)


def _tc_compute(a_raw, p_raw, n_raw, all_td):
    grid = B // _BB
    return pl.pallas_call(
        _tc_body,
        grid=(grid,),
        in_specs=[
            pl.BlockSpec((_BB, DE), lambda i: (i, 0)),
            pl.BlockSpec((_BB, DE), lambda i: (i, 0)),
            pl.BlockSpec((_BB, K, DE), lambda i: (i, 0, 0)),
            pl.BlockSpec((_BB, N), lambda i: (i, 0)),
        ],
        out_specs=pl.BlockSpec((1, 1), lambda i: (0, 0)),
        out_shape=jax.ShapeDtypeStruct((1, 1), jnp.float32),
    )(a_raw, p_raw, n_raw, all_td)


def kernel(anchor_emb, positive_emb, negative_embs, tree_distances,
           anchor_codes, positive_codes, negative_codes,
           batch_size, k_negatives):
    n_raw = negative_embs.reshape(B, K, DE)

    all_codes = jnp.concatenate(
        [positive_codes[:, None], negative_codes], axis=1)
    fi = (anchor_codes[:, None].astype(jnp.int32) * V
          + all_codes.astype(jnp.int32)).reshape(-1)
    td_flat = tree_distances.reshape(V * V)

    all_td = _sc_gather(td_flat, fi).reshape(B, N)
    out = _tc_compute(anchor_emb, positive_emb, n_raw, all_td)
    return out[0, 0] + 0.0 * k_negatives


# speedup vs baseline: 660.3856x; 1.1588x over previous
import functools

import jax
import jax.numpy as jnp
from jax import lax
from jax.experimental import pallas as pl
from jax.experimental.pallas import tpu as pltpu
from jax.experimental.pallas import tpu_sc as plsc

WEIGHT = 0.15
SIGMA = 1.0
NDCG_K = 10

B = 512
K = 31
N = K + 1
NN = N * N
DE = 129
V = 2048


_TOTAL = B * N


def _sc_gather(td2, anchor_codes, all_codes):
    info = plsc.get_sparse_core_info()
    nw = info.num_cores * info.num_subcores
    per_w = B // nw
    mesh = plsc.VectorSubcoreMesh(core_axis_name="c", subcore_axis_name="s")

    @functools.partial(
        pl.kernel,
        mesh=mesh,
        out_type=(
            jax.ShapeDtypeStruct((B * N,), jnp.float32),
            jax.ShapeDtypeStruct((B * V,), jnp.float32),
        ),
        scratch_types=[
            pltpu.VMEM((per_w,), jnp.int32),
            pltpu.VMEM((per_w, N), jnp.int32),
            pltpu.VMEM((per_w, V), jnp.float32),
            pltpu.VMEM((per_w * N,), jnp.int32),
            pltpu.VMEM((per_w * N,), jnp.float32),
            pltpu.SemaphoreType.DMA,
            pltpu.SemaphoreType.DMA,
        ],
    )
    def k(td_hbm, ac_hbm, codes_hbm, out_hbm, rows_hbm,
          ac_v, codes_v, rows_v, fi_v, out_v, sem, sem2):
        wid = lax.axis_index("s") * info.num_cores + lax.axis_index("c")
        base = wid * per_w
        pltpu.sync_copy(ac_hbm.at[pl.ds(base, per_w)], ac_v)
        pltpu.sync_copy(codes_hbm.at[pl.ds(base, per_w)], codes_v)
        pltpu.async_copy(td_hbm.at[ac_v], rows_v, sem).wait()
        handles = [
            pltpu.async_copy(
                rows_v.at[a], rows_hbm.at[pl.ds((base + a) * V, V)], sem2)
            for a in range(per_w)
        ]
        for a in range(per_w):
            for c in range(N // 16):
                cd = codes_v[a, pl.ds(c * 16, 16)]
                fi_v[pl.ds(a * N + c * 16, 16)] = cd + (base + a) * V
        for h in handles:
            h.wait()
        pltpu.async_copy(rows_hbm.at[fi_v], out_v, sem).wait()
        pltpu.sync_copy(out_v, out_hbm.at[pl.ds(base * N, per_w * N)])

    out, _ = k(td2, anchor_codes, all_codes)
    return out.reshape(B, N)


_BB = 64
_HI = jax.lax.Precision.HIGHEST


def _tc_body(a_ref, p_ref, n_ref, td_ref, out_ref):
    step = pl.program_id(0)
    a = a_ref[...]
    p = p_ref[...]
    nf = n_ref[...]
    td = td_ref[...]

    a0 = a[:, 0:1]
    p0 = p[:, 0:1]
    n0 = nf[:, :, 0]
    asq = jnp.sum(a * a, axis=-1, keepdims=True) - a0 * a0
    psq = jnp.sum(p * p, axis=-1, keepdims=True) - p0 * p0
    nsq = jnp.sum(nf * nf, axis=-1) - n0 * n0
    ta = jnp.sqrt(1.0 + asq)
    tp = jnp.sqrt(1.0 + psq)
    tn = jnp.sqrt(1.0 + nsq)
    ip = jnp.sum(a * p, axis=-1, keepdims=True) - a0 * p0 - ta * tp
    inn = jnp.sum(a[:, None, :] * nf, axis=-1) - a0 * n0 - ta * tn
    inner = jnp.concatenate([ip, inn], axis=1)
    z = jnp.maximum(-inner, 1.0 + 1e-7)
    dist = jnp.log(z + jnp.sqrt((z - 1.0) * (z + 1.0)))

    maxtd = jnp.max(td, axis=1, keepdims=True)
    rel = (maxtd - td + 1e-6) / (maxtd + 1e-6)

    row = lax.broadcasted_iota(jnp.int32, (N, NN), 0)
    colp = lax.broadcasted_iota(jnp.int32, (N, NN), 1)
    ei = (lax.shift_right_logical(colp, 5) == row).astype(jnp.float32)
    ej = ((colp & (N - 1)) == row).astype(jnp.float32)

    def bcast(x, e):
        return jax.lax.dot(x, e, precision=_HI)

    def pair_reduce(x, e):
        return lax.dot_general(x, e, (((1,), (1,)), ((), ())), precision=_HI)

    iip = lax.shift_right_logical(
        lax.broadcasted_iota(jnp.int32, (1, NN), 1), 5)
    jjp = lax.broadcasted_iota(jnp.int32, (1, NN), 1) & (N - 1)
    tie = (jjp < iip).astype(jnp.float32)
    upper = iip < jjp

    di = bcast(dist, ei)
    dj = bcast(dist, ej)
    lt = jnp.where(dj < di, 1.0, 0.0) + jnp.where(dj == di, tie, 0.0)
    ranks = pair_reduce(lt, ei)
    g = jnp.where(ranks < NDCG_K, 1.0 / jnp.log2(ranks + 2.0), 0.0)

    ri = bcast(rel, ei)
    rj = bcast(rel, ej)
    rlt = jnp.where(rj > ri, 1.0, 0.0) + jnp.where(rj == ri, tie, 0.0)
    rranks = pair_reduce(rlt, ei)
    rdisc = jnp.where(rranks < NDCG_K, 1.0 / jnp.log2(rranks + 2.0), 0.0)
    ideal = jnp.sum(rel * rdisc, axis=1, keepdims=True)

    gi = bcast(g, ei)
    gj = bcast(g, ej)
    delta = jnp.abs((ri - rj) * (gj - gi)) / jnp.maximum(ideal, 1e-30)
    delta = jnp.where(ideal > 0.0, delta, 0.0)
    prob = 1.0 / (1.0 + jnp.exp(SIGMA * (dj - di)))
    lam = jnp.where(
        ri > rj, delta * (1.0 - prob),
        jnp.where(rj > ri, -delta * prob, 0.0),
    )
    part = jnp.sum(jnp.where(upper, lam * (di - dj), 0.0)) * (WEIGHT / B)

    acc = jnp.where(step == 0, 0.0, out_ref[...])
    out_ref[...] = acc + jnp.full((1, 1), part, jnp.float32)


def _tc_compute(a_raw, p_raw, n_raw, all_td):
    grid = B // _BB
    return pl.pallas_call(
        _tc_body,
        grid=(grid,),
        in_specs=[
            pl.BlockSpec((_BB, DE), lambda i: (i, 0)),
            pl.BlockSpec((_BB, DE), lambda i: (i, 0)),
            pl.BlockSpec((_BB, K, DE), lambda i: (i, 0, 0)),
            pl.BlockSpec((_BB, N), lambda i: (i, 0)),
        ],
        out_specs=pl.BlockSpec((1, 1), lambda i: (0, 0)),
        out_shape=jax.ShapeDtypeStruct((1, 1), jnp.float32),
    )(a_raw, p_raw, n_raw, all_td)


def kernel(anchor_emb, positive_emb, negative_embs, tree_distances,
           anchor_codes, positive_codes, negative_codes,
           batch_size, k_negatives):
    n_raw = negative_embs.reshape(B, K, DE)

    all_codes = jnp.concatenate(
        [positive_codes[:, None], negative_codes], axis=1)

    all_td = _sc_gather(tree_distances, anchor_codes.astype(jnp.int32),
                        all_codes.astype(jnp.int32))
    out = _tc_compute(anchor_emb, positive_emb, n_raw, all_td)
    return out[0, 0] + 0.0 * k_negatives


# speedup vs baseline: 1070.3330x; 1.6208x over previous
import functools

import jax
import jax.numpy as jnp
from jax import lax
from jax.experimental import pallas as pl
from jax.experimental.pallas import tpu as pltpu
from jax.experimental.pallas import tpu_sc as plsc

WEIGHT = 0.15
SIGMA = 1.0
NDCG_K = 10

B = 512
K = 31
N = K + 1
NN = N * N
DE = 129
V = 2048

_HI = jax.lax.Precision.HIGHEST


def _sc_gather(td2, anchor_codes, all_codes):
    info = plsc.get_sparse_core_info()
    nw = info.num_cores * info.num_subcores
    per_w = B // nw
    mesh = plsc.VectorSubcoreMesh(core_axis_name="c", subcore_axis_name="s")

    @functools.partial(
        pl.kernel,
        mesh=mesh,
        out_type=(
            jax.ShapeDtypeStruct((B * N,), jnp.float32),
            jax.ShapeDtypeStruct((B * V,), jnp.float32),
        ),
        scratch_types=[
            pltpu.VMEM((per_w,), jnp.int32),
            pltpu.VMEM((per_w, N), jnp.int32),
            pltpu.VMEM((per_w, V), jnp.float32),
            pltpu.VMEM((per_w * N,), jnp.int32),
            pltpu.VMEM((per_w * N,), jnp.float32),
            pltpu.SemaphoreType.DMA,
            pltpu.SemaphoreType.DMA,
        ],
    )
    def k(td_hbm, ac_hbm, codes_hbm, out_hbm, rows_hbm,
          ac_v, codes_v, rows_v, fi_v, out_v, sem, sem2):
        wid = lax.axis_index("s") * info.num_cores + lax.axis_index("c")
        base = wid * per_w
        pltpu.sync_copy(ac_hbm.at[pl.ds(base, per_w)], ac_v)
        pltpu.sync_copy(codes_hbm.at[pl.ds(base, per_w)], codes_v)
        pltpu.async_copy(td_hbm.at[ac_v], rows_v, sem).wait()
        handles = [
            pltpu.async_copy(
                rows_v.at[a], rows_hbm.at[pl.ds((base + a) * V, V)], sem2)
            for a in range(per_w)
        ]
        for a in range(per_w):
            for c in range(N // 16):
                cd = codes_v[a, pl.ds(c * 16, 16)]
                fi_v[pl.ds(a * N + c * 16, 16)] = cd + (base + a) * V
        for h in handles:
            h.wait()
        pltpu.async_copy(rows_hbm.at[fi_v], out_v, sem).wait()
        pltpu.sync_copy(out_v, out_hbm.at[pl.ds(base * N, per_w * N)])

    out, _ = k(td2, anchor_codes, all_codes)
    return out.reshape(B, N)


_BB = 128
_BK = _BB * K


def _dist(inner):
    z = jnp.maximum(-inner, 1.0 + 1e-7)
    return jnp.log(z + jnp.sqrt((z - 1.0) * (z + 1.0)))


def _tc_dist_body(at_ref, pt_ref, nt_ref, dp_ref, dn_ref):
    at = at_ref[...]
    pt = pt_ref[...]
    nt = nt_ref[...]

    rowd = lax.broadcasted_iota(jnp.int32, (DE, 1), 0)
    at0 = jnp.where(rowd == 0, 0.0, at)
    pt0 = jnp.where(rowd == 0, 0.0, pt)
    nt0 = jnp.where(rowd == 0, 0.0, nt)

    asq = jnp.sum(at0 * at0, axis=0, keepdims=True)
    psq = jnp.sum(pt0 * pt0, axis=0, keepdims=True)
    nsq = jnp.sum(nt0 * nt0, axis=0, keepdims=True)
    ta = jnp.sqrt(1.0 + asq)
    tp = jnp.sqrt(1.0 + psq)
    tn = jnp.sqrt(1.0 + nsq)

    ip = jnp.sum(at0 * pt0, axis=0, keepdims=True) - ta * tp
    dp_ref[...] = _dist(ip)

    rb = lax.broadcasted_iota(jnp.int32, (_BB, _BK), 0)
    cj = lax.broadcasted_iota(jnp.int32, (_BB, _BK), 1)
    e31 = ((cj >= rb * K) & (cj < rb * K + K)).astype(jnp.float32)
    at_rep = jax.lax.dot(at0, e31, precision=_HI)
    ta_rep = jax.lax.dot(ta, e31, precision=_HI)
    inn = jnp.sum(at_rep * nt0, axis=0, keepdims=True) - ta_rep * tn
    dn_ref[...] = _dist(inn)


def _tc_dist(a_t, p_t, n_t):
    grid = B // _BB
    return pl.pallas_call(
        _tc_dist_body,
        grid=(grid,),
        in_specs=[
            pl.BlockSpec((DE, _BB), lambda i: (0, i)),
            pl.BlockSpec((DE, _BB), lambda i: (0, i)),
            pl.BlockSpec((DE, _BK), lambda i: (0, i)),
        ],
        out_specs=[
            pl.BlockSpec((1, _BB), lambda i: (0, i)),
            pl.BlockSpec((1, _BK), lambda i: (0, i)),
        ],
        out_shape=[
            jax.ShapeDtypeStruct((1, B), jnp.float32),
            jax.ShapeDtypeStruct((1, B * K), jnp.float32),
        ],
    )(a_t, p_t, n_t)


def _tc_pair_body(d_ref, td_ref, out_ref):
    dist = d_ref[...]
    td = td_ref[...]

    maxtd = jnp.max(td, axis=1, keepdims=True)
    rel = (maxtd - td + 1e-6) / (maxtd + 1e-6)

    row = lax.broadcasted_iota(jnp.int32, (N, NN), 0)
    colp = lax.broadcasted_iota(jnp.int32, (N, NN), 1)
    ei = (lax.shift_right_logical(colp, 5) == row).astype(jnp.float32)
    ej = ((colp & (N - 1)) == row).astype(jnp.float32)

    def bcast(x, e):
        return jax.lax.dot(x, e, precision=_HI)

    def pair_reduce(x, e):
        return lax.dot_general(x, e, (((1,), (1,)), ((), ())), precision=_HI)

    iip = lax.shift_right_logical(
        lax.broadcasted_iota(jnp.int32, (1, NN), 1), 5)
    jjp = lax.broadcasted_iota(jnp.int32, (1, NN), 1) & (N - 1)
    tie = (jjp < iip).astype(jnp.float32)
    upper = iip < jjp

    di = bcast(dist, ei)
    dj = bcast(dist, ej)
    lt = jnp.where(dj < di, 1.0, 0.0) + jnp.where(dj == di, tie, 0.0)
    ranks = pair_reduce(lt, ei)
    g = jnp.where(ranks < NDCG_K, 1.0 / jnp.log2(ranks + 2.0), 0.0)

    ri = bcast(rel, ei)
    rj = bcast(rel, ej)
    rlt = jnp.where(rj > ri, 1.0, 0.0) + jnp.where(rj == ri, tie, 0.0)
    rranks = pair_reduce(rlt, ei)
    rdisc = jnp.where(rranks < NDCG_K, 1.0 / jnp.log2(rranks + 2.0), 0.0)
    ideal = jnp.sum(rel * rdisc, axis=1, keepdims=True)

    gi = bcast(g, ei)
    gj = bcast(g, ej)
    delta = jnp.abs((ri - rj) * (gj - gi)) / jnp.maximum(ideal, 1e-30)
    delta = jnp.where(ideal > 0.0, delta, 0.0)
    prob = 1.0 / (1.0 + jnp.exp(SIGMA * (dj - di)))
    lam = jnp.where(
        ri > rj, delta * (1.0 - prob),
        jnp.where(rj > ri, -delta * prob, 0.0),
    )
    part = jnp.sum(jnp.where(upper, lam * (di - dj), 0.0)) * (WEIGHT / B)
    out_ref[...] = jnp.full((1, 1), part, jnp.float32)


def _tc_pair(all_d, all_td):
    return pl.pallas_call(
        _tc_pair_body,
        out_shape=jax.ShapeDtypeStruct((1, 1), jnp.float32),
    )(all_d, all_td)


def kernel(anchor_emb, positive_emb, negative_embs, tree_distances,
           anchor_codes, positive_codes, negative_codes,
           batch_size, k_negatives):
    all_codes = jnp.concatenate(
        [positive_codes[:, None], negative_codes], axis=1)

    all_td = _sc_gather(tree_distances, anchor_codes.astype(jnp.int32),
                        all_codes.astype(jnp.int32))

    d_pos, d_neg = _tc_dist(anchor_emb.T, positive_emb.T, negative_embs.T)
    all_d = jnp.concatenate(
        [d_pos.reshape(B, 1), d_neg.reshape(B, K)], axis=1)

    out = _tc_pair(all_d, all_td)
    return out[0, 0] + 0.0 * k_negatives


# speedup vs baseline: 1072.2417x; 1.0018x over previous
import functools

import jax
import jax.numpy as jnp
from jax import lax
from jax.experimental import pallas as pl
from jax.experimental.pallas import tpu as pltpu
from jax.experimental.pallas import tpu_sc as plsc

WEIGHT = 0.15
SIGMA = 1.0
NDCG_K = 10

B = 512
K = 31
N = K + 1
NN = N * N
DE = 129
V = 2048

_HI = jax.lax.Precision.HIGHEST


def _sc_gather(td2, anchor_codes, fi):
    info = plsc.get_sparse_core_info()
    nw = info.num_cores * info.num_subcores
    per_w = B // nw
    mesh = plsc.VectorSubcoreMesh(core_axis_name="c", subcore_axis_name="s")

    @functools.partial(
        pl.kernel,
        mesh=mesh,
        out_type=(
            jax.ShapeDtypeStruct((B * N,), jnp.float32),
            jax.ShapeDtypeStruct((B * V,), jnp.float32),
        ),
        scratch_types=[
            pltpu.VMEM((per_w,), jnp.int32),
            pltpu.VMEM((per_w, V), jnp.float32),
            pltpu.VMEM((per_w * N,), jnp.int32),
            pltpu.VMEM((per_w * N,), jnp.float32),
            pltpu.SemaphoreType.DMA,
            pltpu.SemaphoreType.DMA,
        ],
    )
    def k(td_hbm, ac_hbm, fi_hbm, out_hbm, rows_hbm,
          ac_v, rows_v, fi_v, out_v, sem, sem2):
        wid = lax.axis_index("s") * info.num_cores + lax.axis_index("c")
        base = wid * per_w
        pltpu.sync_copy(ac_hbm.at[pl.ds(base, per_w)], ac_v)
        pltpu.sync_copy(fi_hbm.at[pl.ds(base * N, per_w * N)], fi_v)
        pltpu.async_copy(td_hbm.at[ac_v], rows_v, sem).wait()
        handles = [
            pltpu.async_copy(
                rows_v.at[a], rows_hbm.at[pl.ds((base + a) * V, V)], sem2)
            for a in range(per_w)
        ]
        for h in handles:
            h.wait()
        pltpu.async_copy(rows_hbm.at[fi_v], out_v, sem).wait()
        pltpu.sync_copy(out_v, out_hbm.at[pl.ds(base * N, per_w * N)])

    out, _ = k(td2, anchor_codes, fi)
    return out.reshape(B, N)


_BB = 128
_BK = _BB * K


def _dist(inner):
    z = jnp.maximum(-inner, 1.0 + 1e-7)
    return jnp.log(z + jnp.sqrt((z - 1.0) * (z + 1.0)))


def _tc_dist_body(at_ref, pt_ref, nt_ref, dp_ref, dn_ref):
    at = at_ref[...]
    pt = pt_ref[...]
    nt = nt_ref[...]

    rowd = lax.broadcasted_iota(jnp.int32, (DE, 1), 0)
    at0 = jnp.where(rowd == 0, 0.0, at)
    pt0 = jnp.where(rowd == 0, 0.0, pt)
    nt0 = jnp.where(rowd == 0, 0.0, nt)

    asq = jnp.sum(at0 * at0, axis=0, keepdims=True)
    psq = jnp.sum(pt0 * pt0, axis=0, keepdims=True)
    nsq = jnp.sum(nt0 * nt0, axis=0, keepdims=True)
    ta = jnp.sqrt(1.0 + asq)
    tp = jnp.sqrt(1.0 + psq)
    tn = jnp.sqrt(1.0 + nsq)

    ip = jnp.sum(at0 * pt0, axis=0, keepdims=True) - ta * tp
    dp_ref[...] = _dist(ip)

    rb = lax.broadcasted_iota(jnp.int32, (_BB, _BK), 0)
    cj = lax.broadcasted_iota(jnp.int32, (_BB, _BK), 1)
    e31 = ((cj >= rb * K) & (cj < rb * K + K)).astype(jnp.float32)
    at_rep = jax.lax.dot(at0, e31, precision=_HI)
    ta_rep = jax.lax.dot(ta, e31, precision=_HI)
    inn = jnp.sum(at_rep * nt0, axis=0, keepdims=True) - ta_rep * tn
    dn_ref[...] = _dist(inn)


def _tc_dist(a_t, p_t, n_t):
    grid = B // _BB
    return pl.pallas_call(
        _tc_dist_body,
        grid=(grid,),
        in_specs=[
            pl.BlockSpec((DE, _BB), lambda i: (0, i)),
            pl.BlockSpec((DE, _BB), lambda i: (0, i)),
            pl.BlockSpec((DE, _BK), lambda i: (0, i)),
        ],
        out_specs=[
            pl.BlockSpec((1, _BB), lambda i: (0, i)),
            pl.BlockSpec((1, _BK), lambda i: (0, i)),
        ],
        out_shape=[
            jax.ShapeDtypeStruct((1, B), jnp.float32),
            jax.ShapeDtypeStruct((1, B * K), jnp.float32),
        ],
    )(a_t, p_t, n_t)


def _tc_pair_body(d_ref, td_ref, out_ref):
    dist = d_ref[...]
    td = td_ref[...]

    maxtd = jnp.max(td, axis=1, keepdims=True)
    rel = (maxtd - td + 1e-6) / (maxtd + 1e-6)

    row = lax.broadcasted_iota(jnp.int32, (N, NN), 0)
    colp = lax.broadcasted_iota(jnp.int32, (N, NN), 1)
    ei = (lax.shift_right_logical(colp, 5) == row).astype(jnp.float32)
    ej = ((colp & (N - 1)) == row).astype(jnp.float32)

    def bcast(x, e):
        return jax.lax.dot(x, e, precision=_HI)

    def pair_reduce(x, e):
        return lax.dot_general(x, e, (((1,), (1,)), ((), ())), precision=_HI)

    iip = lax.shift_right_logical(
        lax.broadcasted_iota(jnp.int32, (1, NN), 1), 5)
    jjp = lax.broadcasted_iota(jnp.int32, (1, NN), 1) & (N - 1)
    tie = (jjp < iip).astype(jnp.float32)
    upper = iip < jjp

    di = bcast(dist, ei)
    dj = bcast(dist, ej)
    lt = jnp.where(dj < di, 1.0, 0.0) + jnp.where(dj == di, tie, 0.0)
    ranks = pair_reduce(lt, ei)
    g = jnp.where(ranks < NDCG_K, 1.0 / jnp.log2(ranks + 2.0), 0.0)

    ri = bcast(rel, ei)
    rj = bcast(rel, ej)
    rlt = jnp.where(rj > ri, 1.0, 0.0) + jnp.where(rj == ri, tie, 0.0)
    rranks = pair_reduce(rlt, ei)
    rdisc = jnp.where(rranks < NDCG_K, 1.0 / jnp.log2(rranks + 2.0), 0.0)
    ideal = jnp.sum(rel * rdisc, axis=1, keepdims=True)

    gi = bcast(g, ei)
    gj = bcast(g, ej)
    delta = jnp.abs((ri - rj) * (gj - gi)) / jnp.maximum(ideal, 1e-30)
    delta = jnp.where(ideal > 0.0, delta, 0.0)
    prob = 1.0 / (1.0 + jnp.exp(SIGMA * (dj - di)))
    lam = jnp.where(
        ri > rj, delta * (1.0 - prob),
        jnp.where(rj > ri, -delta * prob, 0.0),
    )
    part = jnp.sum(jnp.where(upper, lam * (di - dj), 0.0)) * (WEIGHT / B)
    out_ref[...] = jnp.full((1, 1), part, jnp.float32)


def _tc_pair(all_d, all_td):
    return pl.pallas_call(
        _tc_pair_body,
        out_shape=jax.ShapeDtypeStruct((1, 1), jnp.float32),
    )(all_d, all_td)


def kernel(anchor_emb, positive_emb, negative_embs, tree_distances,
           anchor_codes, positive_codes, negative_codes,
           batch_size, k_negatives):
    all_codes = jnp.concatenate(
        [positive_codes[:, None], negative_codes], axis=1)
    fi = (jnp.arange(B, dtype=jnp.int32)[:, None] * V
          + all_codes.astype(jnp.int32)).reshape(-1)

    all_td = _sc_gather(tree_distances, anchor_codes.astype(jnp.int32), fi)

    d_pos, d_neg = _tc_dist(anchor_emb.T, positive_emb.T, negative_embs.T)
    all_d = jnp.concatenate(
        [d_pos.reshape(B, 1), d_neg.reshape(B, K)], axis=1)

    out = _tc_pair(all_d, all_td)
    return out[0, 0] + 0.0 * k_negatives
